# baseline (speedup 1.0000x reference)
import functools

import jax
import jax.numpy as jnp
from jax import lax
from jax.experimental import pallas as pl
from jax.experimental.pallas import tpu as pltpu
from jax.experimental.pallas import tpu_sc as plsc

N, P2, W2, C_KV, TOPK = 4, 64, 64, 256, 8
SUB = 8
D = (W2 * C_KV) // SUB
ROWS = N * P2 * TOPK
NW = 32
RPW = ROWS // NW
LANES = 16


def _sc_body(ridx_h, w_h, table_h, out_h, idx_v, w_v, idx_sr, buf, obuf, sem):
    wid = lax.axis_index("s") * 2 + lax.axis_index("c")
    b = wid // (NW // N)
    base_row = pl.multiple_of(wid * RPW, RPW)

    pltpu.sync_copy(ridx_h.at[pl.ds(base_row, RPW)], idx_v)
    pltpu.sync_copy(w_h.at[pl.ds(base_row, RPW)], w_v)

    iota = lax.iota(jnp.int32, LANES)
    hi = lax.div(iota, jnp.int32(SUB))
    lo = lax.rem(iota, jnp.int32(SUB))
    for k in range(RPW // 2):
        g = plsc.load_gather(idx_v, [hi + jnp.int32(2 * k)])
        idx_sr[pl.ds(k * LANES, LANES)] = (g + jnp.int32(b * P2)) * jnp.int32(SUB) + lo

    def do_row(c, carry):
        idxs = idx_sr.at[pl.ds(pl.multiple_of(c * SUB, SUB), SUB)]
        pltpu.async_copy(table_h.at[idxs], buf, sem).wait()
        wvec = plsc.load_gather(w_v, [jnp.full((LANES,), 0, jnp.int32) + c])
        for sr in range(SUB):
            def mul_body(v, cc):
                off = pl.multiple_of(v * LANES, LANES)
                obuf[sr, pl.ds(off, LANES)] = buf[sr, pl.ds(off, LANES)] * wvec
                return cc
            lax.fori_loop(0, D // LANES, mul_body, 0, unroll=8)
        dst = pl.multiple_of(base_row * SUB + c * SUB, SUB)
        pltpu.sync_copy(obuf, out_h.at[pl.ds(dst, SUB)])
        return carry

    lax.fori_loop(0, RPW, do_row, 0)


@jax.jit
def _sc_gather(ridx_flat, w_flat, table):
    mesh = plsc.VectorSubcoreMesh(core_axis_name="c", subcore_axis_name="s")
    k = pl.kernel(
        _sc_body,
        out_type=jax.ShapeDtypeStruct((ROWS * SUB, D), jnp.float32),
        mesh=mesh,
        scratch_types=[
            pltpu.VMEM((RPW,), jnp.int32),
            pltpu.VMEM((RPW,), jnp.float32),
            pltpu.VMEM((RPW * SUB,), jnp.int32),
            pltpu.VMEM((SUB, D), jnp.float32),
            pltpu.VMEM((SUB, D), jnp.float32),
            pltpu.SemaphoreType.DMA,
        ],
        compiler_params=pltpu.CompilerParams(needs_layout_passes=False),
    )
    return k(ridx_flat, w_flat, table)


def kernel(r_idx, r_weight, kv):
    ridx_flat = r_idx.reshape(ROWS)
    w_flat = r_weight.reshape(ROWS)
    table = kv.reshape(ROWS, D)
    out = _sc_gather(ridx_flat, w_flat, table)
    return out.reshape(N, P2, TOPK, W2, C_KV)


# speedup vs baseline: 2.2583x; 2.2583x over previous
import jax
import jax.numpy as jnp
from jax import lax
from jax.experimental import pallas as pl
from jax.experimental.pallas import tpu as pltpu
from jax.experimental.pallas import tpu_sc as plsc

N, P2, W2, C_KV, TOPK = 4, 64, 64, 256, 8
SUB = 8
D = (W2 * C_KV) // SUB
ROWS = N * P2 * TOPK
NW = 32
RPW = ROWS // NW
LANES = 16


def _sc_body(ridx_h, w_h, table_h, out_h,
             idx_v, w_v, idx_sr, in0, in1, ou0, ou1, sg0, sg1, so0, so1):
    wid = lax.axis_index("s") * 2 + lax.axis_index("c")
    b = wid // (NW // N)
    base_row = pl.multiple_of(wid * RPW, RPW)

    ins, outs = (in0, in1), (ou0, ou1)
    sgs, sos = (sg0, sg1), (so0, so1)

    pltpu.sync_copy(ridx_h.at[pl.ds(base_row, RPW)], idx_v)
    pltpu.sync_copy(w_h.at[pl.ds(base_row, RPW)], w_v)

    iota = lax.iota(jnp.int32, LANES)
    hi = lax.div(iota, jnp.int32(SUB))
    lo = lax.rem(iota, jnp.int32(SUB))
    for k in range(RPW // 2):
        g = plsc.load_gather(idx_v, [hi + jnp.int32(2 * k)])
        idx_sr[pl.ds(k * LANES, LANES)] = (g + jnp.int32(b * P2)) * jnp.int32(SUB) + lo

    def start_gather(c, slot):
        idxs = idx_sr.at[pl.ds(pl.multiple_of(c * SUB, SUB), SUB)]
        pltpu.async_copy(table_h.at[idxs], ins[slot], sgs[slot])

    def wait_gather(slot):
        pltpu.make_async_copy(table_h.at[pl.ds(0, SUB)], ins[slot], sgs[slot]).wait()

    def start_out(c, slot):
        dst = pl.multiple_of((base_row + c) * SUB, SUB)
        pltpu.async_copy(outs[slot], out_h.at[pl.ds(dst, SUB)], sos[slot])

    def wait_out(slot):
        pltpu.make_async_copy(outs[slot], out_h.at[pl.ds(0, SUB)], sos[slot]).wait()

    start_gather(0, 0)
    start_gather(1, 1)

    def group(g, carry):
        for slot in range(2):
            c = g * 2 + slot
            wait_gather(slot)

            @pl.when(g > 0)
            def _():
                wait_out(slot)

            wvec = plsc.load_gather(w_v, [jnp.full((LANES,), 0, jnp.int32) + c])
            inb, oub = ins[slot], outs[slot]
            for sr in range(SUB):
                @plsc.parallel_loop(0, D // LANES, unroll=8)
                def _(v):
                    off = pl.multiple_of(v * LANES, LANES)
                    oub[sr, pl.ds(off, LANES)] = inb[sr, pl.ds(off, LANES)] * wvec

            @pl.when(c < RPW - 2)
            def _():
                start_gather(c + 2, slot)
            start_out(c, slot)
        return carry

    lax.fori_loop(0, RPW // 2, group, 0)
    wait_out(0)
    wait_out(1)


@jax.jit
def _sc_gather(ridx_flat, w_flat, table):
    mesh = plsc.VectorSubcoreMesh(core_axis_name="c", subcore_axis_name="s")
    k = pl.kernel(
        _sc_body,
        out_type=jax.ShapeDtypeStruct((ROWS * SUB, D), jnp.float32),
        mesh=mesh,
        scratch_types=[
            pltpu.VMEM((RPW,), jnp.int32),
            pltpu.VMEM((RPW,), jnp.float32),
            pltpu.VMEM((RPW * SUB,), jnp.int32),
            pltpu.VMEM((SUB, D), jnp.float32),
            pltpu.VMEM((SUB, D), jnp.float32),
            pltpu.VMEM((SUB, D), jnp.float32),
            pltpu.VMEM((SUB, D), jnp.float32),
            pltpu.SemaphoreType.DMA,
            pltpu.SemaphoreType.DMA,
            pltpu.SemaphoreType.DMA,
            pltpu.SemaphoreType.DMA,
        ],
        compiler_params=pltpu.CompilerParams(needs_layout_passes=False),
    )
    return k(ridx_flat, w_flat, table)


def kernel(r_idx, r_weight, kv):
    ridx_flat = r_idx.reshape(ROWS)
    w_flat = r_weight.reshape(ROWS)
    table = kv.reshape(ROWS, D)
    out = _sc_gather(ridx_flat, w_flat, table)
    return out.reshape(N, P2, TOPK, W2, C_KV)


# speedup vs baseline: 2.2754x; 1.0075x over previous
import jax
import jax.numpy as jnp
from jax import lax
from jax.experimental import pallas as pl
from jax.experimental.pallas import tpu as pltpu
from jax.experimental.pallas import tpu_sc as plsc

N, P2, W2, C_KV, TOPK = 4, 64, 64, 256, 8
SUB = 8
SL, MIN = 16, 128
ROWS = N * P2 * TOPK
NW = 32
RPW = ROWS // NW
LANES = 16
SLAB_VECS = SL * MIN // LANES


def _sc_body(ridx_h, w_h, table_h, out_h,
             idx_v, w_v, idx_sr, in0, in1, ou0, ou1, sg0, sg1, so0, so1):
    wid = lax.axis_index("s") * 2 + lax.axis_index("c")
    b = wid // (NW // N)
    base_row = pl.multiple_of(wid * RPW, RPW)

    ins, outs = (in0, in1), (ou0, ou1)
    sgs, sos = (sg0, sg1), (so0, so1)

    pltpu.sync_copy(ridx_h.at[pl.ds(base_row, RPW)], idx_v)
    pltpu.sync_copy(w_h.at[pl.ds(base_row, RPW)], w_v)

    iota = lax.iota(jnp.int32, LANES)
    hi = lax.div(iota, jnp.int32(SUB))
    lo = lax.rem(iota, jnp.int32(SUB))
    for k in range(RPW // 2):
        g = plsc.load_gather(idx_v, [hi + jnp.int32(2 * k)])
        idx_sr[pl.ds(k * LANES, LANES)] = (g + jnp.int32(b * P2)) * jnp.int32(SUB) + lo

    def start_gather(c, slot):
        idxs = idx_sr.at[pl.ds(pl.multiple_of(c * SUB, SUB), SUB)]
        pltpu.async_copy(table_h.at[idxs], ins[slot], sgs[slot])

    def wait_gather(slot):
        pltpu.make_async_copy(table_h.at[pl.ds(0, SUB)], ins[slot], sgs[slot]).wait()

    def start_out(c, slot):
        dst = pl.multiple_of((base_row + c) * SUB, SUB)
        pltpu.async_copy(outs[slot], out_h.at[pl.ds(dst, SUB)], sos[slot])

    def wait_out(slot):
        pltpu.make_async_copy(outs[slot], out_h.at[pl.ds(0, SUB)], sos[slot]).wait()

    start_gather(0, 0)
    start_gather(1, 1)

    def group(g, carry):
        for slot in range(2):
            c = g * 2 + slot
            wait_gather(slot)

            @pl.when(g > 0)
            def _():
                wait_out(slot)

            wvec = plsc.load_gather(w_v, [jnp.full((LANES,), 0, jnp.int32) + c])
            inb, oub = ins[slot], outs[slot]
            for a in range(SUB):
                @plsc.parallel_loop(0, SLAB_VECS, unroll=8)
                def _(v):
                    sl = lax.div(v, jnp.int32(MIN // LANES))
                    off = pl.multiple_of(
                        lax.rem(v, jnp.int32(MIN // LANES)) * LANES, LANES)
                    oub[a, sl, pl.ds(off, LANES)] = inb[a, sl, pl.ds(off, LANES)] * wvec

            @pl.when(c < RPW - 2)
            def _():
                start_gather(c + 2, slot)
            start_out(c, slot)
        return carry

    lax.fori_loop(0, RPW // 2, group, 0)
    wait_out(0)
    wait_out(1)


@jax.jit
def _sc_gather(ridx_flat, w_flat, table):
    mesh = plsc.VectorSubcoreMesh(core_axis_name="c", subcore_axis_name="s")
    k = pl.kernel(
        _sc_body,
        out_type=jax.ShapeDtypeStruct((ROWS * SUB, SL, MIN), jnp.float32),
        mesh=mesh,
        scratch_types=[
            pltpu.VMEM((RPW,), jnp.int32),
            pltpu.VMEM((RPW,), jnp.float32),
            pltpu.VMEM((RPW * SUB,), jnp.int32),
            pltpu.VMEM((SUB, SL, MIN), jnp.float32),
            pltpu.VMEM((SUB, SL, MIN), jnp.float32),
            pltpu.VMEM((SUB, SL, MIN), jnp.float32),
            pltpu.VMEM((SUB, SL, MIN), jnp.float32),
            pltpu.SemaphoreType.DMA,
            pltpu.SemaphoreType.DMA,
            pltpu.SemaphoreType.DMA,
            pltpu.SemaphoreType.DMA,
        ],
        compiler_params=pltpu.CompilerParams(
            needs_layout_passes=False,
            use_tc_tiling_on_sc=True,
        ),
    )
    return k(ridx_flat, w_flat, table)


def kernel(r_idx, r_weight, kv):
    ridx_flat = r_idx.reshape(ROWS)
    w_flat = r_weight.reshape(ROWS)
    table = kv.reshape(ROWS, SL, MIN)
    out = _sc_gather(ridx_flat, w_flat, table)
    return out.reshape(N, P2, TOPK, W2, C_KV)


# speedup vs baseline: 5.4641x; 2.4014x over previous
import jax
import jax.numpy as jnp
from jax import lax
from jax.experimental import pallas as pl
from jax.experimental.pallas import tpu as pltpu
from jax.experimental.pallas import tpu_sc as plsc

N, P2, W2, C_KV, TOPK = 4, 64, 64, 256, 8
ROWS = N * P2 * TOPK
REGIONS = N * P2
NW = 32
RPW = ROWS // NW
LANES = 16
ROW_VECS = W2 * C_KV // LANES


def _sc_body(ridx_h, w_h, table_h, out_h,
             idx_v, w_v, idx_stage, in0, in1, ou0, ou1, sg0, sg1, so0, so1):
    wid = lax.axis_index("s") * 2 + lax.axis_index("c")
    b = wid // (NW // N)
    base_row = pl.multiple_of(wid * RPW, RPW)

    ins, outs = (in0, in1), (ou0, ou1)
    sgs, sos = (sg0, sg1), (so0, so1)

    pltpu.sync_copy(ridx_h.at[pl.ds(base_row, RPW)], idx_v)
    pltpu.sync_copy(w_h.at[pl.ds(base_row, RPW)], w_v)

    iota = lax.iota(jnp.int32, LANES)
    lane0 = iota == 0

    def start_gather(c, slot):
        reg = plsc.load_gather(idx_v, [jnp.full((LANES,), 0, jnp.int32) + c])
        reg = reg + jnp.int32(b * P2)
        plsc.store_scatter(idx_stage, [iota * 0 + jnp.int32(8 * slot)], reg,
                           mask=lane0)
        idxs = idx_stage.at[pl.ds(8 * slot, 1)]
        pltpu.async_copy(table_h.at[idxs], ins[slot], sgs[slot])

    def wait_gather(slot):
        pltpu.make_async_copy(table_h.at[pl.ds(0, 1)], ins[slot], sgs[slot]).wait()

    def start_out(c, slot):
        dst = pl.multiple_of(base_row + c, 1)
        pltpu.async_copy(outs[slot], out_h.at[pl.ds(dst, 1)], sos[slot])

    def wait_out(slot):
        pltpu.make_async_copy(outs[slot], out_h.at[pl.ds(0, 1)], sos[slot]).wait()

    start_gather(0, 0)
    start_gather(1, 1)

    def group(g, carry):
        for slot in range(2):
            c = g * 2 + slot
            wait_gather(slot)

            @pl.when(g > 0)
            def _():
                wait_out(slot)

            wvec = plsc.load_gather(w_v, [jnp.full((LANES,), 0, jnp.int32) + c])
            inb, oub = ins[slot], outs[slot]

            @plsc.parallel_loop(0, ROW_VECS, unroll=8)
            def _(v):
                r = lax.div(v, jnp.int32(C_KV // LANES))
                off = pl.multiple_of(
                    lax.rem(v, jnp.int32(C_KV // LANES)) * LANES, LANES)
                oub[0, r, pl.ds(off, LANES)] = inb[0, r, pl.ds(off, LANES)] * wvec

            @pl.when(c < RPW - 2)
            def _():
                start_gather(c + 2, slot)
            start_out(c, slot)
        return carry

    lax.fori_loop(0, RPW // 2, group, 0)
    wait_out(0)
    wait_out(1)


@jax.jit
def _sc_gather(ridx_flat, w_flat, table):
    mesh = plsc.VectorSubcoreMesh(core_axis_name="c", subcore_axis_name="s")
    k = pl.kernel(
        _sc_body,
        out_type=jax.ShapeDtypeStruct((ROWS, W2, C_KV), jnp.float32),
        mesh=mesh,
        scratch_types=[
            pltpu.VMEM((RPW,), jnp.int32),
            pltpu.VMEM((RPW,), jnp.float32),
            pltpu.VMEM((LANES,), jnp.int32),
            pltpu.VMEM((1, W2, C_KV), jnp.float32),
            pltpu.VMEM((1, W2, C_KV), jnp.float32),
            pltpu.VMEM((1, W2, C_KV), jnp.float32),
            pltpu.VMEM((1, W2, C_KV), jnp.float32),
            pltpu.SemaphoreType.DMA,
            pltpu.SemaphoreType.DMA,
            pltpu.SemaphoreType.DMA,
            pltpu.SemaphoreType.DMA,
        ],
        compiler_params=pltpu.CompilerParams(
            needs_layout_passes=False,
            use_tc_tiling_on_sc=True,
        ),
    )
    return k(ridx_flat, w_flat, table)


def kernel(r_idx, r_weight, kv):
    ridx_flat = r_idx.reshape(ROWS)
    w_flat = r_weight.reshape(ROWS)
    table = kv.reshape(REGIONS, W2, C_KV)
    out = _sc_gather(ridx_flat, w_flat, table)
    return out.reshape(N, P2, TOPK, W2, C_KV)


# speedup vs baseline: 5.4777x; 1.0025x over previous
import jax
import jax.numpy as jnp
from jax import lax
from jax.experimental import pallas as pl
from jax.experimental.pallas import tpu as pltpu
from jax.experimental.pallas import tpu_sc as plsc

N, P2, W2, C_KV, TOPK = 4, 64, 64, 256, 8
ROWS = N * P2 * TOPK
REGIONS = N * P2
NW = 32
RPW = ROWS // NW
LANES = 16
ROW_VECS = W2 * C_KV // LANES


NSLOT = 3


def _sc_body(ridx_h, w_h, table_h, out_h,
             idx_v, w_v, idx_stage, in0, in1, in2, ou0, ou1, ou2,
             sg0, sg1, sg2, so0, so1, so2):
    wid = lax.axis_index("s") * 2 + lax.axis_index("c")
    b = wid // (NW // N)
    base_row = pl.multiple_of(wid * RPW, RPW)

    ins, outs = (in0, in1, in2), (ou0, ou1, ou2)
    sgs, sos = (sg0, sg1, sg2), (so0, so1, so2)

    pltpu.sync_copy(ridx_h.at[pl.ds(base_row, RPW)], idx_v)
    pltpu.sync_copy(w_h.at[pl.ds(base_row, RPW)], w_v)

    iota = lax.iota(jnp.int32, LANES)
    lane0 = iota == 0

    def start_gather(c, slot):
        reg = plsc.load_gather(idx_v, [jnp.full((LANES,), 0, jnp.int32) + c])
        reg = reg + jnp.int32(b * P2)
        plsc.store_scatter(idx_stage, [iota * 0 + jnp.int32(8 * slot)], reg,
                           mask=lane0)
        idxs = idx_stage.at[pl.ds(8 * slot, 1)]
        pltpu.async_copy(table_h.at[idxs], ins[slot], sgs[slot])

    def wait_gather(slot):
        pltpu.make_async_copy(table_h.at[pl.ds(0, 1)], ins[slot], sgs[slot]).wait()

    def start_out(c, slot):
        dst = pl.multiple_of(base_row + c, 1)
        pltpu.async_copy(outs[slot], out_h.at[pl.ds(dst, 1)], sos[slot])

    def wait_out(slot):
        pltpu.make_async_copy(outs[slot], out_h.at[pl.ds(0, 1)], sos[slot]).wait()

    for s in range(NSLOT):
        start_gather(s, s)

    def step(c, slot):
        wait_gather(slot)

        @pl.when(c >= NSLOT)
        def _():
            wait_out(slot)

        wvec = plsc.load_gather(w_v, [jnp.full((LANES,), 0, jnp.int32) + c])
        inb, oub = ins[slot], outs[slot]

        @plsc.parallel_loop(0, W2, unroll=2)
        def _(r):
            for h in range(C_KV // LANES):
                oub[0, r, pl.ds(h * LANES, LANES)] = (
                    inb[0, r, pl.ds(h * LANES, LANES)] * wvec)

        @pl.when(c < RPW - NSLOT)
        def _():
            start_gather(c + NSLOT, slot)
        start_out(c, slot)

    def group(g, carry):
        for slot in range(NSLOT):
            step(g * NSLOT + slot, slot)
        return carry

    lax.fori_loop(0, RPW // NSLOT, group, 0)
    for c in range((RPW // NSLOT) * NSLOT, RPW):
        step(jnp.int32(c), c % NSLOT)
    for s in range(NSLOT):
        wait_out(s)


@jax.jit
def _sc_gather(ridx_flat, w_flat, table):
    mesh = plsc.VectorSubcoreMesh(core_axis_name="c", subcore_axis_name="s")
    k = pl.kernel(
        _sc_body,
        out_type=jax.ShapeDtypeStruct((ROWS, W2, C_KV), jnp.float32),
        mesh=mesh,
        scratch_types=[
            pltpu.VMEM((RPW,), jnp.int32),
            pltpu.VMEM((RPW,), jnp.float32),
            pltpu.VMEM((8 * NSLOT,), jnp.int32),
            pltpu.VMEM((1, W2, C_KV), jnp.float32),
            pltpu.VMEM((1, W2, C_KV), jnp.float32),
            pltpu.VMEM((1, W2, C_KV), jnp.float32),
            pltpu.VMEM((1, W2, C_KV), jnp.float32),
            pltpu.VMEM((1, W2, C_KV), jnp.float32),
            pltpu.VMEM((1, W2, C_KV), jnp.float32),
            pltpu.SemaphoreType.DMA,
            pltpu.SemaphoreType.DMA,
            pltpu.SemaphoreType.DMA,
            pltpu.SemaphoreType.DMA,
            pltpu.SemaphoreType.DMA,
            pltpu.SemaphoreType.DMA,
        ],
        compiler_params=pltpu.CompilerParams(
            needs_layout_passes=False,
            use_tc_tiling_on_sc=True,
        ),
    )
    return k(ridx_flat, w_flat, table)


def kernel(r_idx, r_weight, kv):
    ridx_flat = r_idx.reshape(ROWS)
    w_flat = r_weight.reshape(ROWS)
    table = kv.reshape(REGIONS, W2, C_KV)
    out = _sc_gather(ridx_flat, w_flat, table)
    return out.reshape(N, P2, TOPK, W2, C_KV)


# speedup vs baseline: 7.2234x; 1.3187x over previous
import jax
import jax.numpy as jnp
from jax import lax
from jax.experimental import pallas as pl
from jax.experimental.pallas import tpu as pltpu
from jax.experimental.pallas import tpu_sc as plsc

N, P2, W2, C_KV, TOPK = 4, 64, 64, 256, 8
ROWS = N * P2 * TOPK
REGIONS = N * P2
RPB = P2 * TOPK
NW = 32
RGW = P2 * N // NW
LANES = 16
CAP = RPB


def _sc_body(ridx_h, w_h, table_h, out_h,
             idx_b, w_b, mrows, oidx, rg0, rg1, ob0, ob1,
             sr0, sr1, so0, so1):
    wid = lax.axis_index("s") * 2 + lax.axis_index("c")
    batch = wid // (NW // N)
    g8 = wid % (NW // N)
    first_local = g8 * RGW
    regbase = batch * P2 + first_local

    regb, obufs = (rg0, rg1), (ob0, ob1)
    srs, sos = (sr0, sr1), (so0, so1)

    pltpu.sync_copy(ridx_h.at[pl.ds(pl.multiple_of(batch * RPB, RPB), RPB)], idx_b)
    pltpu.sync_copy(w_h.at[pl.ds(pl.multiple_of(batch * RPB, RPB), RPB)], w_b)

    def start_reg(r, slot):
        pltpu.async_copy(table_h.at[pl.ds(regbase + r, 1)], regb[slot], srs[slot])

    def wait_reg(slot):
        pltpu.make_async_copy(table_h.at[pl.ds(0, 1)], regb[slot], srs[slot]).wait()

    def wait_out(slot):
        pltpu.make_async_copy(obufs[slot], out_h.at[pl.ds(0, 1)], sos[slot]).wait()

    start_reg(0, 0)
    start_reg(1, 1)

    iota = lax.iota(jnp.int32, LANES)
    lane0 = iota == 0
    zero16 = jnp.full((LANES,), 0, jnp.int32)

    counts = []
    for R in range(RGW):
        def prep_body(v, cnt):
            ids = idx_b[pl.ds(pl.multiple_of(v * LANES, LANES), LANES)]
            m = ids == jnp.int32(first_local + R)
            pos = cnt + plsc.cumsum(jnp.where(m, 1, 0)) - 1
            plsc.store_scatter(mrows, [jnp.int32(R * CAP) + pos],
                               iota + v * LANES, mask=m)
            return cnt + plsc.all_reduce_population_count(m)
        cnt_vec = lax.fori_loop(0, RPB // LANES, prep_body, zero16)
        counts.append(jnp.max(cnt_vec))

    uses = [jnp.int32(0), jnp.int32(0)]

    for R in range(RGW):
        wait_reg(R % 2)
        reg = regb[R % 2]
        cnt_r = counts[R]

        def make_pair_body(R, reg, cnt_r):
            def pair_body(j2, u):
                u0, u1 = u
                for k in range(2):
                    jj = j2 * 2 + k
                    valid = jj < cnt_r
                    uk = u0 if k == 0 else u1

                    @pl.when(valid)
                    def _():
                        @pl.when(uk > 0)
                        def _():
                            wait_out(k)
                        t = plsc.load_gather(
                            mrows, [zero16 + (jnp.int32(R * CAP) + jj)])
                        wv = plsc.load_gather(w_b, [t])
                        row = t + jnp.int32(batch * RPB)
                        plsc.store_scatter(oidx, [zero16 + k, zero16],
                                           row, mask=lane0)
                        oub = obufs[k]

                        @plsc.parallel_loop(0, W2, unroll=2)
                        def _(r):
                            for h in range(C_KV // LANES):
                                oub[0, r, pl.ds(h * LANES, LANES)] = (
                                    reg[0, r, pl.ds(h * LANES, LANES)] * wv)

                        pltpu.async_copy(oub, out_h.at[oidx.at[k]], sos[k])

                    inc = jnp.where(valid, 1, 0).astype(jnp.int32)
                    if k == 0:
                        u0 = u0 + inc
                    else:
                        u1 = u1 + inc
                return (u0, u1)
            return pair_body

        n_pairs = (cnt_r + 1) // 2
        uses = list(lax.fori_loop(0, n_pairs, make_pair_body(R, reg, cnt_r),
                                  (uses[0], uses[1])))
        if R + 2 < RGW:
            start_reg(R + 2, R % 2)

    for k in range(2):
        @pl.when(uses[k] > 0)
        def _():
            wait_out(k)


@jax.jit
def _sc_gather(ridx_flat, w_flat, table):
    mesh = plsc.VectorSubcoreMesh(core_axis_name="c", subcore_axis_name="s")
    k = pl.kernel(
        _sc_body,
        out_type=jax.ShapeDtypeStruct((ROWS, W2, C_KV), jnp.float32),
        mesh=mesh,
        scratch_types=[
            pltpu.VMEM((RPB,), jnp.int32),
            pltpu.VMEM((RPB,), jnp.float32),
            pltpu.VMEM((RGW * CAP,), jnp.int32),
            pltpu.VMEM((2, 1), jnp.int32),
            pltpu.VMEM((1, W2, C_KV), jnp.float32),
            pltpu.VMEM((1, W2, C_KV), jnp.float32),
            pltpu.VMEM((1, W2, C_KV), jnp.float32),
            pltpu.VMEM((1, W2, C_KV), jnp.float32),
            pltpu.SemaphoreType.DMA,
            pltpu.SemaphoreType.DMA,
            pltpu.SemaphoreType.DMA,
            pltpu.SemaphoreType.DMA,
        ],
        compiler_params=pltpu.CompilerParams(
            needs_layout_passes=False,
            use_tc_tiling_on_sc=True,
        ),
    )
    return k(ridx_flat, w_flat, table)


def kernel(r_idx, r_weight, kv):
    ridx_flat = r_idx.reshape(ROWS)
    w_flat = r_weight.reshape(ROWS)
    table = kv.reshape(REGIONS, W2, C_KV)
    out = _sc_gather(ridx_flat, w_flat, table)
    return out.reshape(N, P2, TOPK, W2, C_KV)
